# baseline (speedup 1.0000x reference)
import functools

import jax
import jax.numpy as jnp
from jax import lax
from jax.experimental import pallas as pl
from jax.experimental.pallas import tpu as pltpu
from jax.experimental.pallas import tpu_sc as plsc

EMB = 8192
DIM = 256
NTOK = 32768
BLK = 512

NC = 2
NS = 16
NW = NC * NS
B_PER_W = NTOK // NW
CHB = 128
NCH = B_PER_W // CHB


def _argmin_body(x_ref, e_ref, fsq_ref, esq_ref, idx_ref):
    simm2 = lax.dot_general(
        x_ref[...], e_ref[...], (((1,), (0,)), ((), ())),
        preferred_element_type=jnp.float32,
    )
    d = (fsq_ref[...] + esq_ref[...]) + simm2
    m = jnp.min(d, axis=1, keepdims=True)
    iota = lax.broadcasted_iota(jnp.int32, d.shape, 1)
    idx_ref[...] = jnp.min(jnp.where(d == m, iota, EMB), axis=1)


@functools.partial(jax.jit, static_argnames=("interpret",))
def _argmin_idx(x_flat, embeddings, interpret=False):
    n = x_flat.shape[0]
    fsq = jnp.sum(x_flat**2, axis=1, keepdims=True)
    esq = jnp.sum(embeddings**2, axis=0, keepdims=True)
    xm2 = x_flat * -2.0
    e_bf = embeddings
    return pl.pallas_call(
        _argmin_body,
        grid=(n // BLK,),
        in_specs=[
            pl.BlockSpec((BLK, DIM), lambda i: (i, 0)),
            pl.BlockSpec((DIM, EMB), lambda i: (0, 0)),
            pl.BlockSpec((BLK, 1), lambda i: (i, 0)),
            pl.BlockSpec((1, EMB), lambda i: (0, 0)),
        ],
        out_specs=pl.BlockSpec((BLK,), lambda i: (i,)),
        out_shape=jax.ShapeDtypeStruct((n,), jnp.int32),
        interpret=interpret,
    )(xm2, e_bf, fsq, esq)


@functools.partial(
    pl.kernel,
    out_type=jax.ShapeDtypeStruct((NTOK, DIM), jnp.float32),
    mesh=plsc.VectorSubcoreMesh(
        core_axis_name="c", subcore_axis_name="s",
        num_cores=NC, num_subcores=NS,
    ),
    scratch_types=[
        pltpu.VMEM((NCH, CHB), jnp.int32),
        pltpu.VMEM((CHB, DIM), jnp.float32),
        pltpu.VMEM((CHB, DIM), jnp.float32),
        pltpu.SemaphoreType.DMA,
        pltpu.SemaphoreType.DMA,
    ],
)
def _sc_gather(table_hbm, idx_hbm, out_hbm, idx_v, rows0, rows1, sem0, sem1):
    wid = lax.axis_index("s") * NC + lax.axis_index("c")
    base = wid * B_PER_W
    pltpu.sync_copy(idx_hbm.at[wid], idx_v)
    rows = (rows0, rows1)
    sems = (sem0, sem1)
    cps = [pltpu.async_copy(table_hbm.at[idx_v.at[0]], rows0, sem0), None]
    for j in range(1, NCH + 1):
        if j < NCH:
            p = j & 1
            cps[p] = pltpu.async_copy(table_hbm.at[idx_v.at[j]], rows[p], sems[p])
        pp = (j - 1) & 1
        cps[pp].wait()
        pltpu.sync_copy(rows[pp], out_hbm.at[pl.ds(base + (j - 1) * CHB, CHB)])


def kernel(x, embeddings):
    x_flat = jnp.reshape(x, (-1, DIM))
    idx = _argmin_idx(x_flat, embeddings)
    table = embeddings.T
    q = _sc_gather(table, jnp.reshape(idx, (NW, NCH, CHB)))
    return jnp.reshape(q, x.shape)


# speedup vs baseline: 1.3809x; 1.3809x over previous
import functools

import jax
import jax.numpy as jnp
from jax import lax
from jax.experimental import pallas as pl
from jax.experimental.pallas import tpu as pltpu
from jax.experimental.pallas import tpu_sc as plsc

EMB = 8192
DIM = 256
NTOK = 32768
BLK = 512

NC = 2
NS = 16
NW = NC * NS
B_PER_W = NTOK // NW
CHB = 128
NCH = B_PER_W // CHB


def _argmin_body(x_ref, e_ref, fsq_ref, esq_ref, idx_ref):
    simm2 = lax.dot_general(
        x_ref[...], e_ref[...], (((1,), (0,)), ((), ())),
        preferred_element_type=jnp.float32,
    )
    fsq = fsq_ref[...]
    esq = esq_ref[...]
    blk = simm2.shape[0]
    LW = 128
    nch = EMB // LW
    mval = (fsq + esq[:, 0:LW]) + simm2[:, 0:LW]
    cidx = jnp.zeros((blk, LW), jnp.int32)
    for k in range(1, nch):
        dk = (fsq + esq[:, k * LW:(k + 1) * LW]) + simm2[:, k * LW:(k + 1) * LW]
        better = dk < mval
        cidx = jnp.where(better, jnp.int32(k), cidx)
        mval = jnp.minimum(mval, dk)
    m = jnp.min(mval, axis=1, keepdims=True)
    lane = lax.broadcasted_iota(jnp.int32, (blk, LW), 1)
    jfull = cidx * LW + lane
    idx_ref[...] = jnp.min(jnp.where(mval == m, jfull, EMB), axis=1)


@functools.partial(jax.jit, static_argnames=("interpret",))
def _argmin_idx(x_flat, embeddings, interpret=False):
    n = x_flat.shape[0]
    fsq = jnp.sum(x_flat**2, axis=1, keepdims=True)
    esq = jnp.sum(embeddings**2, axis=0, keepdims=True)
    xm2 = x_flat * -2.0
    e_bf = embeddings
    return pl.pallas_call(
        _argmin_body,
        grid=(n // BLK,),
        in_specs=[
            pl.BlockSpec((BLK, DIM), lambda i: (i, 0)),
            pl.BlockSpec((DIM, EMB), lambda i: (0, 0)),
            pl.BlockSpec((BLK, 1), lambda i: (i, 0)),
            pl.BlockSpec((1, EMB), lambda i: (0, 0)),
        ],
        out_specs=pl.BlockSpec((BLK,), lambda i: (i,)),
        out_shape=jax.ShapeDtypeStruct((n,), jnp.int32),
        interpret=interpret,
    )(xm2, e_bf, fsq, esq)


@functools.cache
def _sc_gather():
    @functools.partial(
        pl.kernel,
        out_type=jax.ShapeDtypeStruct((NTOK, DIM), jnp.float32),
        mesh=plsc.VectorSubcoreMesh(
            core_axis_name="c", subcore_axis_name="s",
            num_cores=NC, num_subcores=NS,
        ),
        scratch_types=[
            pltpu.VMEM((NCH, CHB), jnp.int32),
            pltpu.VMEM((CHB, DIM), jnp.float32),
            pltpu.VMEM((CHB, DIM), jnp.float32),
            pltpu.SemaphoreType.DMA,
            pltpu.SemaphoreType.DMA,
        ],
    )
    def gather(table_hbm, idx_hbm, out_hbm, idx_v, rows0, rows1, sem0, sem1):
        wid = lax.axis_index("s") * NC + lax.axis_index("c")
        base = wid * B_PER_W
        pltpu.sync_copy(idx_hbm.at[wid], idx_v)
        rows = (rows0, rows1)
        sems = (sem0, sem1)
        cps = [pltpu.async_copy(table_hbm.at[idx_v.at[0]], rows0, sem0), None]
        for j in range(1, NCH + 1):
            if j < NCH:
                p = j & 1
                cps[p] = pltpu.async_copy(table_hbm.at[idx_v.at[j]], rows[p], sems[p])
            pp = (j - 1) & 1
            cps[pp].wait()
            pltpu.sync_copy(rows[pp], out_hbm.at[pl.ds(base + (j - 1) * CHB, CHB)])

    return gather


def kernel(x, embeddings):
    x_flat = jnp.reshape(x, (-1, DIM))
    idx = _argmin_idx(x_flat, embeddings)
    table = embeddings.T
    q = _sc_gather()(table, jnp.reshape(idx, (NW, NCH, CHB)))
    return jnp.reshape(q, x.shape)


# speedup vs baseline: 1.3924x; 1.0083x over previous
import functools

import jax
import jax.numpy as jnp
from jax import lax
from jax.experimental import pallas as pl
from jax.experimental.pallas import tpu as pltpu
from jax.experimental.pallas import tpu_sc as plsc

EMB = 8192
DIM = 256
NTOK = 32768
BLK = 512

NC = 2
NS = 16
NW = NC * NS
B_PER_W = NTOK // NW
CHB = 128
NCH = B_PER_W // CHB


def _argmin_body(x_ref, e_ref, fsq_ref, esq_ref, idx_ref):
    simm2 = lax.dot_general(
        x_ref[...], e_ref[...], (((1,), (0,)), ((), ())),
        preferred_element_type=jnp.float32,
    )
    fsq = fsq_ref[...]
    esq = esq_ref[...]
    blk = simm2.shape[0]
    LW = 128
    TR = 64
    nch = EMB // LW
    lane = lax.broadcasted_iota(jnp.int32, (TR, LW), 1)
    for t in range(blk // TR):
        r0, r1 = t * TR, (t + 1) * TR
        fsq_t = fsq[r0:r1, :]
        mval = (fsq_t + esq[:, 0:LW]) + simm2[r0:r1, 0:LW]
        cidx = jnp.zeros((TR, LW), jnp.int32)
        for k in range(1, nch):
            dk = (fsq_t + esq[:, k * LW:(k + 1) * LW]) + simm2[r0:r1, k * LW:(k + 1) * LW]
            better = dk < mval
            cidx = jnp.where(better, jnp.int32(k), cidx)
            mval = jnp.minimum(mval, dk)
        m = jnp.min(mval, axis=1, keepdims=True)
        jfull = cidx * LW + lane
        idx_ref[r0:r1] = jnp.min(jnp.where(mval == m, jfull, EMB), axis=1)


@functools.partial(jax.jit, static_argnames=("interpret",))
def _argmin_idx(x_flat, embeddings, interpret=False):
    n = x_flat.shape[0]
    fsq = jnp.sum(x_flat**2, axis=1, keepdims=True)
    esq = jnp.sum(embeddings**2, axis=0, keepdims=True)
    xm2 = (x_flat * -2.0).astype(jnp.bfloat16)
    e_bf = embeddings.astype(jnp.bfloat16)
    return pl.pallas_call(
        _argmin_body,
        grid=(n // BLK,),
        in_specs=[
            pl.BlockSpec((BLK, DIM), lambda i: (i, 0)),
            pl.BlockSpec((DIM, EMB), lambda i: (0, 0)),
            pl.BlockSpec((BLK, 1), lambda i: (i, 0)),
            pl.BlockSpec((1, EMB), lambda i: (0, 0)),
        ],
        out_specs=pl.BlockSpec((BLK,), lambda i: (i,)),
        out_shape=jax.ShapeDtypeStruct((n,), jnp.int32),
        interpret=interpret,
    )(xm2, e_bf, fsq, esq)


@functools.cache
def _sc_gather():
    @functools.partial(
        pl.kernel,
        out_type=jax.ShapeDtypeStruct((NTOK, DIM), jnp.float32),
        mesh=plsc.VectorSubcoreMesh(
            core_axis_name="c", subcore_axis_name="s",
            num_cores=NC, num_subcores=NS,
        ),
        scratch_types=[
            pltpu.VMEM((NCH, CHB), jnp.int32),
            pltpu.VMEM((CHB, DIM), jnp.float32),
            pltpu.VMEM((CHB, DIM), jnp.float32),
            pltpu.SemaphoreType.DMA,
            pltpu.SemaphoreType.DMA,
        ],
    )
    def gather(table_hbm, idx_hbm, out_hbm, idx_v, rows0, rows1, sem0, sem1):
        wid = lax.axis_index("s") * NC + lax.axis_index("c")
        base = wid * B_PER_W
        pltpu.sync_copy(idx_hbm.at[wid], idx_v)
        rows = (rows0, rows1)
        sems = (sem0, sem1)
        cps = [pltpu.async_copy(table_hbm.at[idx_v.at[0]], rows0, sem0), None]
        for j in range(1, NCH + 1):
            if j < NCH:
                p = j & 1
                cps[p] = pltpu.async_copy(table_hbm.at[idx_v.at[j]], rows[p], sems[p])
            pp = (j - 1) & 1
            cps[pp].wait()
            pltpu.sync_copy(rows[pp], out_hbm.at[pl.ds(base + (j - 1) * CHB, CHB)])

    return gather


def kernel(x, embeddings):
    x_flat = jnp.reshape(x, (-1, DIM))
    idx = _argmin_idx(x_flat, embeddings)
    table = embeddings.T
    q = _sc_gather()(table, jnp.reshape(idx, (NW, NCH, CHB)))
    return jnp.reshape(q, x.shape)
